# baseline (speedup 1.0000x reference)
import jax
import jax.numpy as jnp
from jax import lax
from jax.experimental import pallas as pl
from jax.experimental.pallas import tpu as pltpu
from jax.experimental.pallas import tpu_sc as plsc

_TB = 10
_ROWS_PER_WORKER = 7


def _sc_flip_body(pos_hbm, out_hbm, sem):
    T = pos_hbm.shape[0]
    w = lax.axis_index("c")
    half = T // 2

    for k in range(half):
        r = w * half + k
        pltpu.async_copy(pos_hbm.at[T - 1 - r], out_hbm.at[r], sem)
    for k in range(half):
        r = w * half + k
        pltpu.make_async_copy(pos_hbm.at[T - 1 - r], out_hbm.at[r], sem).wait()


def _tc_add_body(s_ref, p_ref, o_ref):
    jt = pl.program_id(0)
    for k in range(_TB):
        prow = p_ref[jt * _TB + k]
        o_ref[k] = s_ref[k] + jnp.broadcast_to(prow, s_ref.shape[1:])


def kernel(sessions, pos_emb):
    B, T, F = sessions.shape
    st = jnp.transpose(sessions, (1, 2, 0))

    mesh = plsc.ScalarSubcoreMesh(axis_name="c", num_cores=2)
    pos_flipped = pl.kernel(
        _sc_flip_body,
        out_type=jax.ShapeDtypeStruct((T, F), jnp.float32),
        mesh=mesh,
        scratch_types=[
            pltpu.SemaphoreType.DMA,
        ],
        compiler_params=pltpu.CompilerParams(
            use_tc_tiling_on_sc=True, needs_layout_passes=False
        ),
    )(pos_emb)

    pos3 = pos_flipped[:, :, None]
    out_t = pl.pallas_call(
        _tc_add_body,
        grid=(T // _TB,),
        in_specs=[
            pl.BlockSpec((_TB, F, B), lambda jt: (jt, 0, 0)),
            pl.BlockSpec((T, F, 1), lambda jt: (0, 0, 0)),
        ],
        out_specs=pl.BlockSpec((_TB, F, B), lambda jt: (jt, 0, 0)),
        out_shape=jax.ShapeDtypeStruct((T, F, B), sessions.dtype),
        compiler_params=pltpu.CompilerParams(
            dimension_semantics=("arbitrary",),
            vmem_limit_bytes=100 * 1024 * 1024,
        ),
    )(st, pos3)
    return jnp.transpose(out_t, (2, 0, 1))


# speedup vs baseline: 1.0063x; 1.0063x over previous
import jax
import jax.numpy as jnp
from jax import lax
from jax.experimental import pallas as pl
from jax.experimental.pallas import tpu as pltpu
from jax.experimental.pallas import tpu_sc as plsc

_TB = 10
_ROWS_PER_WORKER = 7


def _sc_flip_body(pos_hbm, out_hbm, rowbuf, sem):
    T = pos_hbm.shape[0]
    w = lax.axis_index("s") * 2 + lax.axis_index("c")

    def each_row(fn):
        for k in range(_ROWS_PER_WORKER):
            r = w * _ROWS_PER_WORKER + k

            def _run(k=k, r=r):
                fn(k, r)

            pl.when(r < T)(_run)

    def fire_read(k, r):
        pltpu.async_copy(pos_hbm.at[T - 1 - r], rowbuf.at[k], sem)

    def drain_read(k, r):
        pltpu.make_async_copy(pos_hbm.at[T - 1 - r], rowbuf.at[k], sem).wait()

    def fire_write(k, r):
        pltpu.async_copy(rowbuf.at[k], out_hbm.at[r], sem)

    def drain_write(k, r):
        pltpu.make_async_copy(rowbuf.at[k], out_hbm.at[r], sem).wait()

    each_row(fire_read)
    each_row(drain_read)
    each_row(fire_write)
    each_row(drain_write)


def _tc_add_body(s_ref, p_ref, o_ref):
    jt = pl.program_id(0)
    for k in range(_TB):
        prow = p_ref[jt * _TB + k]
        o_ref[k] = s_ref[k] + jnp.broadcast_to(prow, s_ref.shape[1:])


def kernel(sessions, pos_emb):
    B, T, F = sessions.shape
    st = jnp.transpose(sessions, (1, 2, 0))

    mesh = plsc.VectorSubcoreMesh(core_axis_name="c", subcore_axis_name="s")
    pos_flipped = pl.kernel(
        _sc_flip_body,
        out_type=jax.ShapeDtypeStruct((T, F), jnp.float32),
        mesh=mesh,
        scratch_types=[
            pltpu.VMEM((_ROWS_PER_WORKER, F), jnp.float32),
            pltpu.SemaphoreType.DMA,
        ],
        compiler_params=pltpu.CompilerParams(
            use_tc_tiling_on_sc=True, needs_layout_passes=False
        ),
    )(pos_emb)

    pos3 = pos_flipped[:, :, None]
    out_t = pl.pallas_call(
        _tc_add_body,
        grid=(T // _TB,),
        in_specs=[
            pl.BlockSpec((_TB, F, B), lambda jt: (jt, 0, 0)),
            pl.BlockSpec((T, F, 1), lambda jt: (0, 0, 0)),
        ],
        out_specs=pl.BlockSpec((_TB, F, B), lambda jt: (jt, 0, 0)),
        out_shape=jax.ShapeDtypeStruct((T, F, B), sessions.dtype),
        compiler_params=pltpu.CompilerParams(
            dimension_semantics=("arbitrary",),
            vmem_limit_bytes=100 * 1024 * 1024,
        ),
    )(st, pos3)
    return jnp.transpose(out_t, (2, 0, 1))
